# baseline (speedup 1.0000x reference)
import functools
import jax
import jax.numpy as jnp
from jax import lax
from jax.experimental import pallas as pl
from jax.experimental.pallas import tpu as pltpu
from jax.experimental.pallas import tpu_sc as plsc

_MAX = 64
_S = 512
_H = 256
_NW = 32
_GI = 128
_CJ = 64
_W = 192


def _sc_body(x_hbm, t_hbm, out_hbm, idx_a, idx_b, xc, dwin, ob0, ob1, ob2,
             ob3, gsem, s0, s1, s2, s3):
    wid = lax.axis_index("s") * 2 + lax.axis_index("c")
    g = wid // 8
    cc = wid % 8
    jbase = cc * _CJ
    ibase = g * _GI
    base_k = 384 - _GI * g + _CJ * cc

    lane = lax.iota(jnp.int32, 16)
    for m0 in range(0, 96, 16):
        v = lane + (m0 + base_k - 511)
        idx_a[pl.ds(m0, 16)] = jnp.clip(v, -_MAX, _MAX) + _MAX
        idx_b[pl.ds(m0, 16)] = jnp.clip(v + 96, -_MAX, _MAX) + _MAX

    pltpu.sync_copy(x_hbm.at[0, pl.ds(jbase, _CJ)], xc)
    pltpu.async_copy(t_hbm.at[idx_a], dwin.at[pl.ds(0, 96)], gsem).wait()
    pltpu.async_copy(t_hbm.at[idx_b], dwin.at[pl.ds(96, 96)], gsem).wait()

    def pair(ba, bb, sa, sb, ia, off, c0, first):
        jb = jbase + c0

        @pl.when(jnp.logical_not(first))
        def _():
            pltpu.make_async_copy(
                xc, out_hbm.at[ia, pl.ds(jbase, _CJ)], sa).wait()

        @plsc.parallel_loop(0, _CJ // 2, unroll=4)
        def _row(jj):
            for v in range(_H // 16):
                sl = pl.ds(v * 16, 16)
                a = xc[c0 + jj, sl]
                ba[jj, sl] = a + dwin[off + c0 + jj, sl]
                bb[jj, sl] = a + dwin[off - 1 + c0 + jj, sl]

        pltpu.async_copy(ba, out_hbm.at[ia, pl.ds(jb, _CJ // 2)], sa)
        pltpu.async_copy(bb, out_hbm.at[ia + 1, pl.ds(jb, _CJ // 2)], sa)

    def step(t, carry):
        ii0 = 2 * t
        i0 = ibase + ii0
        off0 = (_GI - 1) - ii0
        first = t == 0
        pair(ob0, ob1, s0, s1, i0, off0, 0, first)
        pair(ob2, ob3, s2, s3, i0, off0, _CJ // 2, first)
        return carry

    lax.fori_loop(0, _GI // 2, step, 0)
    pltpu.make_async_copy(xc, out_hbm.at[ibase, pl.ds(jbase, _CJ)], s0).wait()
    pltpu.make_async_copy(xc, out_hbm.at[ibase, pl.ds(jbase, _CJ)], s2).wait()


def kernel(x, table):
    mesh = plsc.VectorSubcoreMesh(core_axis_name="c", subcore_axis_name="s")
    run = pl.kernel(
        _sc_body,
        out_type=jax.ShapeDtypeStruct((_S, _S, _H), jnp.float32),
        mesh=mesh,
        scratch_types=[
            pltpu.VMEM((96,), jnp.int32),
            pltpu.VMEM((96,), jnp.int32),
            pltpu.VMEM((_CJ, _H), jnp.float32),
            pltpu.VMEM((_W, _H), jnp.float32),
            pltpu.VMEM((_CJ // 2, _H), jnp.float32),
            pltpu.VMEM((_CJ // 2, _H), jnp.float32),
            pltpu.VMEM((_CJ // 2, _H), jnp.float32),
            pltpu.VMEM((_CJ // 2, _H), jnp.float32),
            pltpu.SemaphoreType.DMA,
            pltpu.SemaphoreType.DMA,
            pltpu.SemaphoreType.DMA,
            pltpu.SemaphoreType.DMA,
            pltpu.SemaphoreType.DMA,
        ],
    )
    return run(x, table)


# speedup vs baseline: 1.0041x; 1.0041x over previous
import jax
import jax.numpy as jnp
from jax import lax
from jax.experimental import pallas as pl
from jax.experimental.pallas import tpu as pltpu
from jax.experimental.pallas import tpu_sc as plsc

_MAX = 64
_S = 512
_H = 256
_GI = 128
_CJ = 64
_W = 192


def _sc_body(x_hbm, t_hbm, out_hbm, idx_a, idx_b, xc, dwin, ob0, ob1, ob2,
             ob3, gsem, s0, s2):
    wid = lax.axis_index("s") * 2 + lax.axis_index("c")
    g = wid // 8
    cc = wid % 8
    jbase = cc * _CJ
    ibase = g * _GI
    base_k = 384 - _GI * g + _CJ * cc

    lane = lax.iota(jnp.int32, 16)
    for m0 in range(0, 96, 16):
        v = lane + (m0 + base_k - 511)
        idx_a[pl.ds(m0, 16)] = jnp.clip(v, -_MAX, _MAX) + _MAX
        idx_b[pl.ds(m0, 16)] = jnp.clip(v + 96, -_MAX, _MAX) + _MAX

    pltpu.sync_copy(x_hbm.at[0, pl.ds(jbase, _CJ)], xc)
    pltpu.async_copy(t_hbm.at[idx_a], dwin.at[pl.ds(0, 96)], gsem).wait()
    pltpu.async_copy(t_hbm.at[idx_b], dwin.at[pl.ds(96, 96)], gsem).wait()

    def pair(ba, bb, sa, ia, off, c0, first):
        jb = jbase + c0

        @pl.when(jnp.logical_not(first))
        def _():
            pltpu.make_async_copy(
                xc, out_hbm.at[ia, pl.ds(jbase, _CJ)], sa).wait()

        @plsc.parallel_loop(0, _CJ // 2, unroll=4)
        def _row(jj):
            for v in range(_H // 16):
                sl = pl.ds(v * 16, 16)
                a = xc[c0 + jj, sl]
                ba[jj, sl] = a + dwin[off + c0 + jj, sl]
                bb[jj, sl] = a + dwin[off - 1 + c0 + jj, sl]

        pltpu.async_copy(ba, out_hbm.at[ia, pl.ds(jb, _CJ // 2)], sa)
        pltpu.async_copy(bb, out_hbm.at[ia + 1, pl.ds(jb, _CJ // 2)], sa)

    def step(t, carry):
        ii0 = 2 * t
        i0 = ibase + ii0
        off0 = (_GI - 1) - ii0
        first = t == 0
        pair(ob0, ob1, s0, i0, off0, 0, first)
        pair(ob2, ob3, s2, i0, off0, _CJ // 2, first)
        return carry

    lax.fori_loop(0, _GI // 2, step, 0)
    pltpu.make_async_copy(xc, out_hbm.at[ibase, pl.ds(jbase, _CJ)], s0).wait()
    pltpu.make_async_copy(xc, out_hbm.at[ibase, pl.ds(jbase, _CJ)], s2).wait()


def kernel(x, table):
    mesh = plsc.VectorSubcoreMesh(core_axis_name="c", subcore_axis_name="s")
    run = pl.kernel(
        _sc_body,
        out_type=jax.ShapeDtypeStruct((_S, _S, _H), jnp.float32),
        mesh=mesh,
        scratch_types=[
            pltpu.VMEM((96,), jnp.int32),
            pltpu.VMEM((96,), jnp.int32),
            pltpu.VMEM((_CJ, _H), jnp.float32),
            pltpu.VMEM((_W, _H), jnp.float32),
            pltpu.VMEM((_CJ // 2, _H), jnp.float32),
            pltpu.VMEM((_CJ // 2, _H), jnp.float32),
            pltpu.VMEM((_CJ // 2, _H), jnp.float32),
            pltpu.VMEM((_CJ // 2, _H), jnp.float32),
            pltpu.SemaphoreType.DMA,
            pltpu.SemaphoreType.DMA,
            pltpu.SemaphoreType.DMA,
        ],
    )
    return run(x, table)


# speedup vs baseline: 1.0205x; 1.0163x over previous
import jax
import jax.numpy as jnp
from jax import lax
from jax.experimental import pallas as pl
from jax.experimental.pallas import tpu as pltpu
from jax.experimental.pallas import tpu_sc as plsc

_MAX = 64
_S = 512
_H = 256
_GI = 128
_CJ = 64
_W = 192


def _sc_body(x_hbm, t_hbm, out_hbm, idx_a, idx_b, xc, dwin, ob0, ob1, ob2,
             ob3, gsem, s0, s2):
    wid = lax.axis_index("s") * 2 + lax.axis_index("c")
    g = wid // 8
    cc = wid % 8
    jbase = cc * _CJ
    ibase = g * _GI
    base_k = 384 - _GI * g + _CJ * cc

    lane = lax.iota(jnp.int32, 16)
    for m0 in range(0, 96, 16):
        v = lane + (m0 + base_k - 511)
        idx_a[pl.ds(m0, 16)] = jnp.clip(v, -_MAX, _MAX) + _MAX
        idx_b[pl.ds(m0, 16)] = jnp.clip(v + 96, -_MAX, _MAX) + _MAX

    h1 = pltpu.async_copy(t_hbm.at[idx_a], dwin.at[pl.ds(0, 96)], gsem)
    h2 = pltpu.async_copy(t_hbm.at[idx_b], dwin.at[pl.ds(96, 96)], gsem)
    pltpu.sync_copy(x_hbm.at[0, pl.ds(jbase, _CJ)], xc)
    h1.wait()
    h2.wait()

    def pair(ba, bb, sa, ia, off, c0, first):
        jb = jbase + c0

        @pl.when(jnp.logical_not(first))
        def _():
            pltpu.make_async_copy(
                xc, out_hbm.at[ia, pl.ds(jbase, _CJ)], sa).wait()

        @plsc.parallel_loop(0, _CJ // 2, unroll=4)
        def _row(jj):
            for v in range(_H // 16):
                sl = pl.ds(v * 16, 16)
                a = xc[c0 + jj, sl]
                ba[jj, sl] = a + dwin[off + c0 + jj, sl]
                bb[jj, sl] = a + dwin[off - 1 + c0 + jj, sl]

        pltpu.async_copy(ba, out_hbm.at[ia, pl.ds(jb, _CJ // 2)], sa)
        pltpu.async_copy(bb, out_hbm.at[ia + 1, pl.ds(jb, _CJ // 2)], sa)

    def step(t, carry):
        ii0 = 2 * t
        i0 = ibase + ii0
        off0 = (_GI - 1) - ii0
        first = t == 0
        pair(ob0, ob1, s0, i0, off0, 0, first)
        pair(ob2, ob3, s2, i0, off0, _CJ // 2, first)
        return carry

    lax.fori_loop(0, _GI // 2, step, 0)
    pltpu.make_async_copy(xc, out_hbm.at[ibase, pl.ds(jbase, _CJ)], s0).wait()
    pltpu.make_async_copy(xc, out_hbm.at[ibase, pl.ds(jbase, _CJ)], s2).wait()


def kernel(x, table):
    mesh = plsc.VectorSubcoreMesh(core_axis_name="c", subcore_axis_name="s")
    run = pl.kernel(
        _sc_body,
        out_type=jax.ShapeDtypeStruct((_S, _S, _H), jnp.float32),
        mesh=mesh,
        scratch_types=[
            pltpu.VMEM((96,), jnp.int32),
            pltpu.VMEM((96,), jnp.int32),
            pltpu.VMEM((_CJ, _H), jnp.float32),
            pltpu.VMEM((_W, _H), jnp.float32),
            pltpu.VMEM((_CJ // 2, _H), jnp.float32),
            pltpu.VMEM((_CJ // 2, _H), jnp.float32),
            pltpu.VMEM((_CJ // 2, _H), jnp.float32),
            pltpu.VMEM((_CJ // 2, _H), jnp.float32),
            pltpu.SemaphoreType.DMA,
            pltpu.SemaphoreType.DMA,
            pltpu.SemaphoreType.DMA,
        ],
    )
    return run(x, table)
